# baseline (speedup 1.0000x reference)
import functools

import jax
import jax.numpy as jnp
from jax import lax
from jax.experimental import pallas as pl
from jax.experimental.pallas import tpu as pltpu
from jax.experimental.pallas import tpu_sc as plsc

D_MODEL = 1024
_NC = 2
_NS = 16
_NW = _NC * _NS
_L = 16


def _team_gather_t(table_t, tail_t, idx):
    B = idx.shape[0]
    D = table_t.shape[0]
    V = table_t.shape[1]
    thresh = (V // 128) * 128
    b_per_w = B // _NW
    mesh = plsc.VectorSubcoreMesh(core_axis_name="c", subcore_axis_name="s")

    K = 4

    @functools.partial(
        pl.kernel,
        mesh=mesh,
        out_type=jax.ShapeDtypeStruct((B, D), jnp.float32),
        scratch_types=[
            pltpu.VMEM((b_per_w,), jnp.int32),
            pltpu.VMEM((K, D, 128), jnp.float32),
            pltpu.VMEM((b_per_w, D), jnp.float32),
            pltpu.SemaphoreType.DMA,
            pltpu.SemaphoreType.DMA,
            pltpu.SemaphoreType.DMA,
            pltpu.SemaphoreType.DMA,
        ],
        compiler_params=pltpu.CompilerParams(use_tc_tiling_on_sc=True,
                                             needs_layout_passes=False),
    )
    def k(table_hbm, tail_hbm, idx_hbm, out_hbm, idx_v, tiles_v, rows_v,
          s0, s1, s2, s3):
        sems = (s0, s1, s2, s3)
        wid = lax.axis_index("s") * _NC + lax.axis_index("c")
        base = wid * b_per_w
        pltpu.sync_copy(idx_hbm.at[pl.ds(base, b_per_w)], idx_v)
        vs = [idx_v[pl.ds(t * _L, _L)] for t in range(b_per_w // _L)]

        def fire(i):
            s = vs[i // _L][i % _L]
            start = pl.multiple_of(
                lax.shift_left(lax.shift_right_logical(s, 7), 7), 128)

            @pl.when(s < thresh)
            def _():
                pltpu.make_async_copy(
                    table_hbm.at[:, pl.ds(start, 128)],
                    tiles_v.at[i % K], sems[i % K]).start()

            @pl.when(s >= thresh)
            def _():
                pltpu.make_async_copy(
                    tail_hbm, tiles_v.at[i % K], sems[i % K]).start()

        def drain(i):
            pltpu.make_async_copy(
                table_hbm.at[:, pl.ds(0, 128)],
                tiles_v.at[i % K], sems[i % K]).wait()

        rlo = lax.iota(jnp.int32, _L)
        rhi = rlo + _L
        for i in range(K):
            fire(i)
        for i in range(b_per_w):
            s = vs[i // _L][i % _L]
            c = jnp.full((_L,), lax.bitwise_and(s, 127), jnp.int32)
            drain(i)
            lo = plsc.load_gather(tiles_v.at[i % K], [rlo, c])
            hi = plsc.load_gather(tiles_v.at[i % K], [rhi, c])
            ivec = jnp.full((_L,), i, jnp.int32)
            plsc.store_scatter(rows_v, [ivec, rlo], lo)
            plsc.store_scatter(rows_v, [ivec, rhi], hi)
            if i + K < b_per_w:
                fire(i + K)
        pltpu.sync_copy(rows_v, out_hbm.at[pl.ds(base, b_per_w)])

    return k(table_t, tail_t, idx)


def _assembly_body(pf_ref, cls_ref, type_ref, out_ref):
    j = pl.program_id(1)
    tv = type_ref[...]

    @pl.when(j == 0)
    def _():
        row = cls_ref[...] + tv[0:1]
        out_ref[...] = jnp.broadcast_to(row[None], out_ref.shape)

    @pl.when(j > 0)
    def _():
        trow = jnp.where(j <= 6, tv[1:2], tv[2:3])
        out_ref[...] = pf_ref[...] + trow[None]


def _field_body(team_ref, w_ref, t_ref, hz_ref, wtab_ref, ttab_ref, Wf_ref,
                b_ref, g_ref, bt_ref, type_ref, acc_ref, out_ref):
    del acc_ref
    TB = w_ref.shape[0]
    dot = functools.partial(jnp.dot, preferred_element_type=jnp.float32,
                            precision=lax.Precision.HIGHEST)
    iota = lax.broadcasted_iota(jnp.int32, (TB, 16), 1)
    w_oh = (w_ref[...] == iota).astype(jnp.float32)
    t_oh = (t_ref[...] == iota).astype(jnp.float32)
    Wf = Wf_ref[...]
    wproj = dot(wtab_ref[...], Wf[32:48, :])
    tproj = dot(ttab_ref[...], Wf[48:64, :])
    Wcat = jnp.concatenate([Wf[0:32, :], wproj, tproj, Wf[64:80, :]], axis=0)
    X = jnp.concatenate([team_ref[...], w_oh, t_oh, hz_ref[...]], axis=1)
    pre = dot(X, Wcat) + b_ref[...]
    mu = jnp.mean(pre, axis=-1, keepdims=True)
    var = jnp.mean((pre - mu) ** 2, axis=-1, keepdims=True)
    ft = (pre - mu) * lax.rsqrt(var + 1e-5) * g_ref[...] + bt_ref[...]
    out_ref[...] = (ft + type_ref[...][3:4])[None]


def kernel(team_id, weather, terrain, hazards, pokemon_features, team_table,
           weather_table, terrain_table, W_field, b_field, ln_gamma, ln_beta,
           cls_token, type_table):
    B = team_id.shape[0]
    V = team_table.shape[0]
    thresh = (V // 128) * 128
    tail = jnp.pad(team_table[thresh:].T, ((0, 0), (0, 128 - (V - thresh))))
    team_emb = _team_gather_t(team_table.T, tail, team_id.astype(jnp.int32))
    pf_t = jnp.transpose(pokemon_features, (1, 0, 2))
    w2 = weather.astype(jnp.int32).reshape(B, 1)
    t2 = terrain.astype(jnp.int32).reshape(B, 1)
    b2 = b_field.reshape(1, D_MODEL)
    g2 = ln_gamma.reshape(1, D_MODEL)
    bt2 = ln_beta.reshape(1, D_MODEL)
    cls2 = cls_token.reshape(1, D_MODEL)

    TB = 512
    full = lambda i, j: (0, 0)
    acc = pl.pallas_call(
        _assembly_body,
        grid=(B // TB, 13),
        in_specs=[
            pl.BlockSpec((1, TB, D_MODEL),
                         lambda i, j: (jnp.maximum(j - 1, 0), i, 0)),
            pl.BlockSpec((1, D_MODEL), full),
            pl.BlockSpec((4, D_MODEL), full),
        ],
        out_specs=pl.BlockSpec((1, TB, D_MODEL), lambda i, j: (j, i, 0)),
        out_shape=jax.ShapeDtypeStruct((14, B, D_MODEL), jnp.float32),
        compiler_params=pltpu.CompilerParams(
            dimension_semantics=("arbitrary", "arbitrary")),
    )(pf_t, cls2, type_table)

    TBF = 1024
    fullf = lambda i: (0, 0)
    out_t = pl.pallas_call(
        _field_body,
        grid=(B // TBF,),
        in_specs=[
            pl.BlockSpec((TBF, 32), lambda i: (i, 0)),
            pl.BlockSpec((TBF, 1), lambda i: (i, 0)),
            pl.BlockSpec((TBF, 1), lambda i: (i, 0)),
            pl.BlockSpec((TBF, 16), lambda i: (i, 0)),
            pl.BlockSpec((16, 16), fullf),
            pl.BlockSpec((16, 16), fullf),
            pl.BlockSpec((96, D_MODEL), fullf),
            pl.BlockSpec((1, D_MODEL), fullf),
            pl.BlockSpec((1, D_MODEL), fullf),
            pl.BlockSpec((1, D_MODEL), fullf),
            pl.BlockSpec((4, D_MODEL), fullf),
            pl.BlockSpec(memory_space=pl.ANY),
        ],
        out_specs=pl.BlockSpec((1, TBF, D_MODEL), lambda i: (13, i, 0)),
        out_shape=jax.ShapeDtypeStruct((14, B, D_MODEL), jnp.float32),
        input_output_aliases={11: 0},
        compiler_params=pltpu.CompilerParams(
            dimension_semantics=("arbitrary",)),
    )(team_emb, w2, t2, hazards, weather_table, terrain_table, W_field,
      b2, g2, bt2, type_table, acc)
    return jnp.transpose(out_t, (1, 0, 2))


# speedup vs baseline: 1.0572x; 1.0572x over previous
import functools

import jax
import jax.numpy as jnp
from jax import lax
from jax.experimental import pallas as pl
from jax.experimental.pallas import tpu as pltpu
from jax.experimental.pallas import tpu_sc as plsc

D_MODEL = 1024
_NC = 2
_NS = 16
_NW = _NC * _NS
_L = 16


def _team_gather_t(table_t, tail_t, idx):
    B = idx.shape[0]
    D = table_t.shape[0]
    V = table_t.shape[1]
    thresh = (V // 128) * 128
    b_per_w = B // _NW
    mesh = plsc.VectorSubcoreMesh(core_axis_name="c", subcore_axis_name="s")

    K = 4

    @functools.partial(
        pl.kernel,
        mesh=mesh,
        out_type=jax.ShapeDtypeStruct((B, D), jnp.float32),
        scratch_types=[
            pltpu.VMEM((b_per_w,), jnp.int32),
            pltpu.VMEM((K, D, 128), jnp.float32),
            pltpu.VMEM((b_per_w, D), jnp.float32),
            pltpu.SemaphoreType.DMA,
            pltpu.SemaphoreType.DMA,
            pltpu.SemaphoreType.DMA,
            pltpu.SemaphoreType.DMA,
        ],
        compiler_params=pltpu.CompilerParams(use_tc_tiling_on_sc=True,
                                             needs_layout_passes=False),
    )
    def k(table_hbm, tail_hbm, idx_hbm, out_hbm, idx_v, tiles_v, rows_v,
          s0, s1, s2, s3):
        sems = (s0, s1, s2, s3)
        wid = lax.axis_index("s") * _NC + lax.axis_index("c")
        base = wid * b_per_w
        pltpu.sync_copy(idx_hbm.at[pl.ds(base, b_per_w)], idx_v)
        vs = [idx_v[pl.ds(t * _L, _L)] for t in range(b_per_w // _L)]

        def fire(i):
            s = vs[i // _L][i % _L]
            start = pl.multiple_of(
                lax.shift_left(lax.shift_right_logical(s, 7), 7), 128)

            @pl.when(s < thresh)
            def _():
                pltpu.make_async_copy(
                    table_hbm.at[:, pl.ds(start, 128)],
                    tiles_v.at[i % K], sems[i % K]).start()

            @pl.when(s >= thresh)
            def _():
                pltpu.make_async_copy(
                    tail_hbm, tiles_v.at[i % K], sems[i % K]).start()

        def drain(i):
            pltpu.make_async_copy(
                table_hbm.at[:, pl.ds(0, 128)],
                tiles_v.at[i % K], sems[i % K]).wait()

        rlo = lax.iota(jnp.int32, _L)
        rhi = rlo + _L
        for i in range(K):
            fire(i)
        for i in range(b_per_w):
            s = vs[i // _L][i % _L]
            c = jnp.full((_L,), lax.bitwise_and(s, 127), jnp.int32)
            drain(i)
            lo = plsc.load_gather(tiles_v.at[i % K], [rlo, c])
            hi = plsc.load_gather(tiles_v.at[i % K], [rhi, c])
            ivec = jnp.full((_L,), i, jnp.int32)
            plsc.store_scatter(rows_v, [ivec, rlo], lo)
            plsc.store_scatter(rows_v, [ivec, rhi], hi)
            if i + K < b_per_w:
                fire(i + K)
        pltpu.sync_copy(rows_v, out_hbm.at[pl.ds(base, b_per_w)])

    return k(table_t, tail_t, idx)


def _assembly_body(pf_ref, cls_ref, type_ref, out_ref):
    j = pl.program_id(1)
    tv = type_ref[...]

    @pl.when(j == 0)
    def _():
        row = cls_ref[...] + tv[0:1]
        out_ref[...] = jnp.broadcast_to(row[None], out_ref.shape)

    @pl.when(j > 0)
    def _():
        trow = jnp.where(j <= 6, tv[1:2], tv[2:3])
        out_ref[...] = pf_ref[...] + trow[None]


def _field_body(team_ref, w_ref, t_ref, hz_ref, wtab_ref, ttab_ref, Wf_ref,
                b_ref, g_ref, bt_ref, type_ref, acc_ref, out_ref):
    del acc_ref
    TB = w_ref.shape[0]
    dot = functools.partial(jnp.dot, preferred_element_type=jnp.float32,
                            precision=lax.Precision.HIGHEST)
    iota = lax.broadcasted_iota(jnp.int32, (TB, 16), 1)
    w_oh = (w_ref[...] == iota).astype(jnp.float32)
    t_oh = (t_ref[...] == iota).astype(jnp.float32)
    Wf = Wf_ref[...]
    wproj = dot(wtab_ref[...], Wf[32:48, :])
    tproj = dot(ttab_ref[...], Wf[48:64, :])
    Wcat = jnp.concatenate([Wf[0:32, :], wproj, tproj, Wf[64:80, :]], axis=0)
    X = jnp.concatenate([team_ref[...], w_oh, t_oh, hz_ref[...]], axis=1)
    pre = dot(X, Wcat) + b_ref[...]
    mu = jnp.mean(pre, axis=-1, keepdims=True)
    var = jnp.mean((pre - mu) ** 2, axis=-1, keepdims=True)
    ft = (pre - mu) * lax.rsqrt(var + 1e-5) * g_ref[...] + bt_ref[...]
    out_ref[...] = (ft + type_ref[...][3:4])[None]


def kernel(team_id, weather, terrain, hazards, pokemon_features, team_table,
           weather_table, terrain_table, W_field, b_field, ln_gamma, ln_beta,
           cls_token, type_table):
    B = team_id.shape[0]
    V = team_table.shape[0]
    thresh = (V // 128) * 128
    tail = jnp.pad(team_table[thresh:].T, ((0, 0), (0, 128 - (V - thresh))))
    team_emb = _team_gather_t(team_table.T, tail, team_id.astype(jnp.int32))
    pf_t = jnp.transpose(pokemon_features, (1, 0, 2))
    w2 = weather.astype(jnp.int32).reshape(B, 1)
    t2 = terrain.astype(jnp.int32).reshape(B, 1)
    b2 = b_field.reshape(1, D_MODEL)
    g2 = ln_gamma.reshape(1, D_MODEL)
    bt2 = ln_beta.reshape(1, D_MODEL)
    cls2 = cls_token.reshape(1, D_MODEL)

    TB = 1024
    full = lambda i, j: (0, 0)
    acc = pl.pallas_call(
        _assembly_body,
        grid=(B // TB, 13),
        in_specs=[
            pl.BlockSpec((1, TB, D_MODEL),
                         lambda i, j: (jnp.maximum(j - 1, 0), i, 0)),
            pl.BlockSpec((1, D_MODEL), full),
            pl.BlockSpec((4, D_MODEL), full),
        ],
        out_specs=pl.BlockSpec((1, TB, D_MODEL), lambda i, j: (j, i, 0)),
        out_shape=jax.ShapeDtypeStruct((14, B, D_MODEL), jnp.float32),
        compiler_params=pltpu.CompilerParams(
            dimension_semantics=("arbitrary", "arbitrary")),
    )(pf_t, cls2, type_table)

    TBF = 512
    fullf = lambda i: (0, 0)
    out_t = pl.pallas_call(
        _field_body,
        grid=(B // TBF,),
        in_specs=[
            pl.BlockSpec((TBF, 32), lambda i: (i, 0)),
            pl.BlockSpec((TBF, 1), lambda i: (i, 0)),
            pl.BlockSpec((TBF, 1), lambda i: (i, 0)),
            pl.BlockSpec((TBF, 16), lambda i: (i, 0)),
            pl.BlockSpec((16, 16), fullf),
            pl.BlockSpec((16, 16), fullf),
            pl.BlockSpec((96, D_MODEL), fullf),
            pl.BlockSpec((1, D_MODEL), fullf),
            pl.BlockSpec((1, D_MODEL), fullf),
            pl.BlockSpec((1, D_MODEL), fullf),
            pl.BlockSpec((4, D_MODEL), fullf),
            pl.BlockSpec(memory_space=pl.ANY),
        ],
        out_specs=pl.BlockSpec((1, TBF, D_MODEL), lambda i: (13, i, 0)),
        out_shape=jax.ShapeDtypeStruct((14, B, D_MODEL), jnp.float32),
        input_output_aliases={11: 0},
        compiler_params=pltpu.CompilerParams(
            dimension_semantics=("arbitrary",)),
    )(team_emb, w2, t2, hazards, weather_table, terrain_table, W_field,
      b2, g2, bt2, type_table, acc)
    return jnp.transpose(out_t, (1, 0, 2))


# speedup vs baseline: 1.0684x; 1.0106x over previous
import functools

import jax
import jax.numpy as jnp
from jax import lax
from jax.experimental import pallas as pl
from jax.experimental.pallas import tpu as pltpu
from jax.experimental.pallas import tpu_sc as plsc

D_MODEL = 1024
_NC = 2
_NS = 16
_NW = _NC * _NS
_L = 16


def _team_gather_t(table_t, tail_t, idx):
    B = idx.shape[0]
    D = table_t.shape[0]
    V = table_t.shape[1]
    thresh = (V // 128) * 128
    b_per_w = B // _NW
    mesh = plsc.VectorSubcoreMesh(core_axis_name="c", subcore_axis_name="s")

    K = 4

    @functools.partial(
        pl.kernel,
        mesh=mesh,
        out_type=jax.ShapeDtypeStruct((B, D), jnp.float32),
        scratch_types=[
            pltpu.VMEM((b_per_w,), jnp.int32),
            pltpu.VMEM((K, D, 128), jnp.float32),
            pltpu.VMEM((b_per_w, D), jnp.float32),
            pltpu.SemaphoreType.DMA,
            pltpu.SemaphoreType.DMA,
            pltpu.SemaphoreType.DMA,
            pltpu.SemaphoreType.DMA,
        ],
        compiler_params=pltpu.CompilerParams(use_tc_tiling_on_sc=True,
                                             needs_layout_passes=False),
    )
    def k(table_hbm, tail_hbm, idx_hbm, out_hbm, idx_v, tiles_v, rows_v,
          s0, s1, s2, s3):
        sems = (s0, s1, s2, s3)
        wid = lax.axis_index("s") * _NC + lax.axis_index("c")
        base = wid * b_per_w
        pltpu.sync_copy(idx_hbm.at[pl.ds(base, b_per_w)], idx_v)
        vs = [idx_v[pl.ds(t * _L, _L)] for t in range(b_per_w // _L)]

        def fire(i):
            s = vs[i // _L][i % _L]
            start = pl.multiple_of(
                lax.shift_left(lax.shift_right_logical(s, 7), 7), 128)

            @pl.when(s < thresh)
            def _():
                pltpu.make_async_copy(
                    table_hbm.at[:, pl.ds(start, 128)],
                    tiles_v.at[i % K], sems[i % K]).start()

            @pl.when(s >= thresh)
            def _():
                pltpu.make_async_copy(
                    tail_hbm, tiles_v.at[i % K], sems[i % K]).start()

        def drain(i):
            pltpu.make_async_copy(
                table_hbm.at[:, pl.ds(0, 128)],
                tiles_v.at[i % K], sems[i % K]).wait()

        rlo = lax.iota(jnp.int32, _L)
        rhi = rlo + _L
        for i in range(K):
            fire(i)
        for i in range(b_per_w):
            s = vs[i // _L][i % _L]
            c = jnp.full((_L,), lax.bitwise_and(s, 127), jnp.int32)
            drain(i)
            lo = plsc.load_gather(tiles_v.at[i % K], [rlo, c])
            hi = plsc.load_gather(tiles_v.at[i % K], [rhi, c])
            ivec = jnp.full((_L,), i, jnp.int32)
            plsc.store_scatter(rows_v, [ivec, rlo], lo)
            plsc.store_scatter(rows_v, [ivec, rhi], hi)
            if i + K < b_per_w:
                fire(i + K)
        pltpu.sync_copy(rows_v, out_hbm.at[pl.ds(base, b_per_w)])

    return k(table_t, tail_t, idx)


def _assembly_body(pf_ref, cls_ref, type_ref, out_ref):
    j = pl.program_id(1)
    tv = type_ref[...]

    @pl.when(j == 0)
    def _():
        row = cls_ref[...] + tv[0:1]
        out_ref[...] = jnp.broadcast_to(row[None], out_ref.shape)

    @pl.when(j > 0)
    def _():
        trow = jnp.where(j <= 6, tv[1:2], tv[2:3])
        out_ref[...] = pf_ref[...] + trow[None]


def _field_body(team_ref, w_ref, t_ref, hz_ref, wtab_ref, ttab_ref, Wf_ref,
                b_ref, g_ref, bt_ref, type_ref, acc_ref, out_ref):
    del acc_ref
    TB = w_ref.shape[0]
    dot = functools.partial(jnp.dot, preferred_element_type=jnp.float32,
                            precision=lax.Precision.HIGHEST)
    iota = lax.broadcasted_iota(jnp.int32, (TB, 16), 1)
    w_oh = (w_ref[...] == iota).astype(jnp.float32)
    t_oh = (t_ref[...] == iota).astype(jnp.float32)
    Wf = Wf_ref[...]
    wproj = dot(wtab_ref[...], Wf[32:48, :])
    tproj = dot(ttab_ref[...], Wf[48:64, :])
    Wcat = jnp.concatenate([Wf[0:32, :], wproj, tproj, Wf[64:80, :]], axis=0)
    X = jnp.concatenate([team_ref[...], w_oh, t_oh, hz_ref[...]], axis=1)
    pre = dot(X, Wcat) + b_ref[...]
    mu = jnp.mean(pre, axis=-1, keepdims=True)
    var = jnp.mean((pre - mu) ** 2, axis=-1, keepdims=True)
    ft = (pre - mu) * lax.rsqrt(var + 1e-5) * g_ref[...] + bt_ref[...]
    out_ref[...] = (ft + type_ref[...][3:4])[None]


def kernel(team_id, weather, terrain, hazards, pokemon_features, team_table,
           weather_table, terrain_table, W_field, b_field, ln_gamma, ln_beta,
           cls_token, type_table):
    B = team_id.shape[0]
    V = team_table.shape[0]
    thresh = (V // 128) * 128
    tail = jnp.pad(team_table[thresh:].T, ((0, 0), (0, 128 - (V - thresh))))
    team_emb = _team_gather_t(team_table.T, tail, team_id.astype(jnp.int32))
    pf_t = jnp.transpose(pokemon_features, (1, 0, 2))
    w2 = weather.astype(jnp.int32).reshape(B, 1)
    t2 = terrain.astype(jnp.int32).reshape(B, 1)
    b2 = b_field.reshape(1, D_MODEL)
    g2 = ln_gamma.reshape(1, D_MODEL)
    bt2 = ln_beta.reshape(1, D_MODEL)
    cls2 = cls_token.reshape(1, D_MODEL)

    TB = 2048
    full = lambda i, j: (0, 0)
    acc = pl.pallas_call(
        _assembly_body,
        grid=(B // TB, 13),
        in_specs=[
            pl.BlockSpec((1, TB, D_MODEL),
                         lambda i, j: (jnp.maximum(j - 1, 0), i, 0)),
            pl.BlockSpec((1, D_MODEL), full),
            pl.BlockSpec((4, D_MODEL), full),
        ],
        out_specs=pl.BlockSpec((1, TB, D_MODEL), lambda i, j: (j, i, 0)),
        out_shape=jax.ShapeDtypeStruct((14, B, D_MODEL), jnp.float32),
        compiler_params=pltpu.CompilerParams(
            dimension_semantics=("arbitrary", "arbitrary")),
    )(pf_t, cls2, type_table)

    TBF = 512
    fullf = lambda i: (0, 0)
    out_t = pl.pallas_call(
        _field_body,
        grid=(B // TBF,),
        in_specs=[
            pl.BlockSpec((TBF, 32), lambda i: (i, 0)),
            pl.BlockSpec((TBF, 1), lambda i: (i, 0)),
            pl.BlockSpec((TBF, 1), lambda i: (i, 0)),
            pl.BlockSpec((TBF, 16), lambda i: (i, 0)),
            pl.BlockSpec((16, 16), fullf),
            pl.BlockSpec((16, 16), fullf),
            pl.BlockSpec((96, D_MODEL), fullf),
            pl.BlockSpec((1, D_MODEL), fullf),
            pl.BlockSpec((1, D_MODEL), fullf),
            pl.BlockSpec((1, D_MODEL), fullf),
            pl.BlockSpec((4, D_MODEL), fullf),
            pl.BlockSpec(memory_space=pl.ANY),
        ],
        out_specs=pl.BlockSpec((1, TBF, D_MODEL), lambda i: (13, i, 0)),
        out_shape=jax.ShapeDtypeStruct((14, B, D_MODEL), jnp.float32),
        input_output_aliases={11: 0},
        compiler_params=pltpu.CompilerParams(
            dimension_semantics=("arbitrary",)),
    )(team_emb, w2, t2, hazards, weather_table, terrain_table, W_field,
      b2, g2, bt2, type_table, acc)
    return jnp.transpose(out_t, (1, 0, 2))
